# baseline (speedup 1.0000x reference)
import functools
import jax
import jax.numpy as jnp
from jax import lax
from jax.experimental import pallas as pl
from jax.experimental.pallas import tpu as pltpu
from jax.experimental.pallas import tpu_sc as plsc

_M = 4096
_N = 8192
_NW = 32
_ROWS_PER_W = _M // _NW
_L = 16
_NV = _N // _L


def _sc_body(x_hbm, out_hbm, row_v, out_v):
    wid = lax.axis_index("s") * 2 + lax.axis_index("c")
    base = wid * _ROWS_PER_W

    def row_loop(r, _):
        row = base + r
        pltpu.sync_copy(x_hbm.at[row], row_v)

        def vreg_loop(i, carry):
            v = row_v[pl.ds(i * _L, _L)]
            incl = jnp.cumsum(v)
            out_v[pl.ds(i * _L, _L)] = incl - v + carry
            return carry + jnp.sum(v)

        lax.fori_loop(0, _NV, vreg_loop, jnp.float32(0.0))
        pltpu.sync_copy(out_v, out_hbm.at[row])
        return 0

    lax.fori_loop(0, _ROWS_PER_W, row_loop, 0)


@jax.jit
def kernel(x):
    mesh = plsc.VectorSubcoreMesh(core_axis_name="c", subcore_axis_name="s")
    f = functools.partial(
        pl.kernel,
        mesh=mesh,
        out_type=jax.ShapeDtypeStruct((_M, _N), jnp.float32),
        scratch_types=[
            pltpu.VMEM((_N,), jnp.float32),
            pltpu.VMEM((_N,), jnp.float32),
        ],
        compiler_params=pltpu.CompilerParams(needs_layout_passes=False),
    )(_sc_body)
    return f(x)


# speedup vs baseline: 5.0513x; 5.0513x over previous
import jax
import jax.numpy as jnp
from jax.experimental import pallas as pl
from jax.experimental.pallas import tpu as pltpu

_R = 256
_C = 8192
_SUB = 128


def _scan_kernel(x_ref, o_ref, carry_ref):
    ci = pl.program_id(0)
    ri = pl.program_id(1)
    rbase = ri * _R

    @pl.when(ci == 0)
    def _():
        carry_ref[pl.ds(rbase, _R), :] = jnp.zeros((_R, _SUB), jnp.float32)

    x = x_ref[...]
    T = (jax.lax.broadcasted_iota(jnp.int32, (_SUB, _SUB), 0)
         < jax.lax.broadcasted_iota(jnp.int32, (_SUB, _SUB), 1)
         ).astype(jnp.float32)
    ones = jnp.ones((_SUB, _SUB), jnp.float32)
    B = jnp.concatenate([T, ones], axis=1)
    carry = carry_ref[pl.ds(rbase, _R), :]
    for k in range(_C // _SUB):
        xs = x[:, k * _SUB:(k + 1) * _SUB]
        y = jnp.dot(xs, B, preferred_element_type=jnp.float32)
        o_ref[:, k * _SUB:(k + 1) * _SUB] = y[:, :_SUB] + carry
        carry = carry + y[:, _SUB:]
    carry_ref[pl.ds(rbase, _R), :] = carry


@jax.jit
def kernel(x):
    m, n = x.shape
    grid = (n // _C, m // _R)
    return pl.pallas_call(
        _scan_kernel,
        grid=grid,
        in_specs=[pl.BlockSpec((_R, _C), lambda j, i: (i, j))],
        out_specs=pl.BlockSpec((_R, _C), lambda j, i: (i, j)),
        out_shape=jax.ShapeDtypeStruct((m, n), x.dtype),
        scratch_shapes=[pltpu.VMEM((m, _SUB), jnp.float32)],
        compiler_params=pltpu.CompilerParams(
            dimension_semantics=("arbitrary", "arbitrary")),
    )(x)
